# baseline (speedup 1.0000x reference)
import functools

import jax
import jax.numpy as jnp
from jax import lax
from jax.experimental import pallas as pl
from jax.experimental.pallas import tpu as pltpu
from jax.experimental.pallas import tpu_sc as plsc

_COMMITMENT = 1.0


def _argmin_body(x_ref, w_ref, b_ref, cb_ref, idx_ref, loss_ref, *, T, K, KC, n_t, scale):
    b = pl.program_id(0)
    t = pl.program_id(1)
    xb = x_ref[0]
    z = lax.dot_general(xb, w_ref[...], (((0,), (0,)), ((), ())))
    z = z + b_ref[...]
    z2 = jnp.sum(z * z, axis=1, keepdims=True)

    mval = None
    midx = None
    for i in range(K // KC):
        cb = cb_ref[pl.ds(i * KC, KC), :]
        s = lax.dot_general(z, cb, (((1,), (1,)), ((), ())))
        c2 = jnp.sum(cb * cb, axis=1)
        d2 = z2 - 2.0 * s + c2[None, :]
        lm = jnp.min(d2, axis=1, keepdims=True)
        io = lax.broadcasted_iota(jnp.int32, (T, KC), 1) + (i * KC)
        li = jnp.min(jnp.where(d2 == lm, io, K), axis=1, keepdims=True)
        if mval is None:
            mval, midx = lm, li
        else:
            take = lm < mval
            mval = jnp.where(take, lm, mval)
            midx = jnp.where(take, li, midx)

    idx_ref[0, 0] = midx[:, 0]

    @pl.when(jnp.logical_and(b == 0, t == 0))
    def _():
        loss_ref[0] = 0.0

    loss_ref[0] += jnp.sum(mval) * scale


def _proj_out_body(q_ref, w_ref, b_ref, out_ref):
    q = q_ref[0]
    o = lax.dot_general(w_ref[...], q, (((0,), (1,)), ((), ())))
    out_ref[0] = o + b_ref[...]


def _sc_gather(codebook, idx_flat):
    K, C = codebook.shape
    N = idx_flat.shape[0]
    info = plsc.get_sparse_core_info()
    NC, NS = info.num_cores, info.num_subcores
    NW = NC * NS
    b_per_w = N // NW
    n_sub = b_per_w // 128
    idx2 = idx_flat.reshape(N // 128, 128)

    mesh = plsc.VectorSubcoreMesh(core_axis_name="c", subcore_axis_name="s")

    @functools.partial(
        pl.kernel,
        out_type=jax.ShapeDtypeStruct((N, C), jnp.float32),
        mesh=mesh,
        scratch_types=[
            pltpu.VMEM((n_sub, 128), jnp.int32),
            pltpu.VMEM((b_per_w, C), jnp.float32),
            pltpu.SemaphoreType.DMA,
        ],
        compiler_params=pltpu.CompilerParams(use_tc_tiling_on_sc=False),
    )
    def gather(cb_hbm, idx_hbm, out_hbm, idx_v, rows_v, sem):
        wid = lax.axis_index("s") * NC + lax.axis_index("c")
        base = wid * b_per_w
        pltpu.sync_copy(idx_hbm.at[pl.ds(wid * n_sub, n_sub)], idx_v)
        cps = [
            pltpu.async_copy(
                cb_hbm.at[idx_v.at[j]],
                rows_v.at[pl.ds(j * 128, 128)],
                sem,
            )
            for j in range(n_sub)
        ]
        for cp in cps:
            cp.wait()
        pltpu.sync_copy(rows_v, out_hbm.at[pl.ds(base, b_per_w)])

    return gather(codebook, idx2)


def _stage_a(x3, W_in, b_in, codebook, *, T, KC):
    B, D, HW = x3.shape
    K, C = codebook.shape
    n_t = HW // T
    N = B * HW
    scale = _COMMITMENT / (N * C)
    body = functools.partial(_argmin_body, T=T, K=K, KC=KC, n_t=n_t, scale=scale)
    return pl.pallas_call(
        body,
        grid=(B, n_t),
        in_specs=[
            pl.BlockSpec((1, D, T), lambda b, t: (b, 0, t)),
            pl.BlockSpec((D, C), lambda b, t: (0, 0)),
            pl.BlockSpec((1, C), lambda b, t: (0, 0)),
            pl.BlockSpec((K, C), lambda b, t: (0, 0)),
        ],
        out_specs=[
            pl.BlockSpec((1, 1, T), lambda b, t: (b * n_t + t, 0, 0)),
            pl.BlockSpec(memory_space=pltpu.MemorySpace.SMEM),
        ],
        out_shape=[
            jax.ShapeDtypeStruct((B * n_t, 1, T), jnp.int32),
            jax.ShapeDtypeStruct((1,), jnp.float32),
        ],
    )(x3, W_in, b_in.reshape(1, C), codebook)


def _stage_c(q3, W_out, b_out, *, T):
    B, HW, C = q3.shape
    D = W_out.shape[1]
    n_t = HW // T
    return pl.pallas_call(
        _proj_out_body,
        grid=(B, n_t),
        in_specs=[
            pl.BlockSpec((1, T, C), lambda b, t: (b, t, 0)),
            pl.BlockSpec((C, D), lambda b, t: (0, 0)),
            pl.BlockSpec((D, 1), lambda b, t: (0, 0)),
        ],
        out_specs=pl.BlockSpec((1, D, T), lambda b, t: (b, 0, t)),
        out_shape=jax.ShapeDtypeStruct((B, D, HW), jnp.float32),
    )(q3, W_out, b_out.reshape(D, 1))


def kernel(x, W_in, b_in, codebook, W_out, b_out):
    B, D, H, W = x.shape
    HW = H * W
    N = B * HW
    K, C = codebook.shape
    T = 256
    KC = 2048

    x3 = x.reshape(B, D, HW)
    idx3, loss = _stage_a(x3, W_in, b_in, codebook, T=T, KC=KC)
    q = _sc_gather(codebook, idx3.reshape(N))
    out3 = _stage_c(q.reshape(B, HW, C), W_out, b_out, T=T)
    out = out3.reshape(B, D, H, W)
    return out, idx3.reshape(B, HW), loss[0]
